# baseline (speedup 1.0000x reference)
import functools

import jax
import jax.numpy as jnp
from jax import lax
from jax.experimental import pallas as pl
from jax.experimental.pallas import tpu as pltpu
from jax.experimental.pallas import tpu_sc as plsc

BATCH = 16384
N_FIELDS = 26
N_FIELDS_PAD = 32
CARD = 64
OUT_W = N_FIELDS * CARD

NUM_WORKERS = 32
ROWS_PER_WORKER = BATCH // NUM_WORKERS
CHUNK_ROWS = 32
CHUNKS = ROWS_PER_WORKER // CHUNK_ROWS
LANES = 16


def _body(x_hbm, out_hbm, x_vmem, buf0, buf1, sem0, sem1, semx):
    c = lax.axis_index("c")
    s = lax.axis_index("s")
    wid = s * 2 + c

    ones = jnp.full((LANES,), 1, jnp.int32)
    zeros = jnp.zeros((LANES,), jnp.int32)

    def _splat(v):
        return jnp.full((LANES,), v, jnp.int32)

    xcopy = pltpu.make_async_copy(
        x_hbm.at[:, pl.ds(wid * ROWS_PER_WORKER, ROWS_PER_WORKER)], x_vmem, semx
    )
    xcopy.start()

    rows0 = lax.iota(jnp.int32, LANES)
    rows1 = rows0 + _splat(LANES)

    def scatter_pass(buf, k, value_vec):
        row0 = k * CHUNK_ROWS

        def ib(f, _):
            colbase = _splat(f * CARD)
            vals0 = x_vmem[f, pl.ds(row0, LANES)]
            plsc.store_scatter(buf, [rows0, colbase + vals0], value_vec)
            vals1 = x_vmem[f, pl.ds(row0 + LANES, LANES)]
            plsc.store_scatter(buf, [rows1, colbase + vals1], value_vec)
            return 0

        lax.fori_loop(0, N_FIELDS, ib, 0)

    rbase = wid * ROWS_PER_WORKER

    def dma(buf, k, sem):
        return pltpu.make_async_copy(
            buf, out_hbm.at[pl.ds(rbase + k * CHUNK_ROWS, CHUNK_ROWS)], sem
        )

    def zero_buf(buf):
        def z(r, _):
            for u in range(OUT_W // LANES):
                buf[r, pl.ds(u * LANES, LANES)] = zeros
            return 0

        lax.fori_loop(0, CHUNK_ROWS, z, 0)

    def zero_half(buf, half):
        def z(r, _):
            for u in range(OUT_W // LANES):
                buf[half * (CHUNK_ROWS // 2) + r, pl.ds(u * LANES, LANES)] = zeros
            return 0

        lax.fori_loop(0, CHUNK_ROWS // 2, z, 0)

    def scatter_half(buf, half, value_vec):
        rows = rows1 if half else rows0

        def ib(f, _):
            vals = x_vmem[f, pl.ds(half * LANES, LANES)]
            plsc.store_scatter(buf, [rows, _splat(f * CARD) + vals], value_vec)
            return 0

        lax.fori_loop(0, N_FIELDS, ib, 0)

    def dma_half(buf, half, sem):
        return pltpu.make_async_copy(
            buf.at[pl.ds(half * (CHUNK_ROWS // 2), CHUNK_ROWS // 2)],
            out_hbm.at[pl.ds(rbase + half * (CHUNK_ROWS // 2), CHUNK_ROWS // 2)],
            sem,
        )

    zero_half(buf0, 0)
    xcopy.wait()
    scatter_half(buf0, 0, ones)
    dma_half(buf0, 0, sem0).start()
    zero_half(buf0, 1)
    scatter_half(buf0, 1, ones)
    dma_half(buf0, 1, sem0).start()
    zero_buf(buf1)
    scatter_pass(buf1, 1, ones)
    dma(buf1, 1, sem1).start()

    def chunk_pair(j, _):
        k0 = 2 * j
        k1 = 2 * j + 1
        dma(buf0, k0 - 2, sem0).wait()
        scatter_pass(buf0, k0 - 2, zeros)
        scatter_pass(buf0, k0, ones)
        dma(buf0, k0, sem0).start()
        dma(buf1, k1 - 2, sem1).wait()
        scatter_pass(buf1, k1 - 2, zeros)
        scatter_pass(buf1, k1, ones)
        dma(buf1, k1, sem1).start()
        return 0

    lax.fori_loop(1, CHUNKS // 2, chunk_pair, 0)
    dma(buf0, CHUNKS - 2, sem0).wait()
    dma(buf1, CHUNKS - 1, sem1).wait()


@jax.jit
def _onehot(x):
    xt = x.T
    mesh = plsc.VectorSubcoreMesh(core_axis_name="c", subcore_axis_name="s")
    f = functools.partial(
        pl.kernel,
        out_type=jax.ShapeDtypeStruct((BATCH, OUT_W), jnp.int32),
        scratch_types=[
            pltpu.VMEM((N_FIELDS, ROWS_PER_WORKER), jnp.int32),
            pltpu.VMEM((CHUNK_ROWS, OUT_W), jnp.int32),
            pltpu.VMEM((CHUNK_ROWS, OUT_W), jnp.int32),
            pltpu.SemaphoreType.DMA,
            pltpu.SemaphoreType.DMA,
            pltpu.SemaphoreType.DMA,
        ],
        mesh=mesh,
        compiler_params=pltpu.CompilerParams(needs_layout_passes=False),
    )(_body)
    return f(xt)


def kernel(x):
    return _onehot(x)


# speedup vs baseline: 1.0207x; 1.0207x over previous
import functools

import jax
import jax.numpy as jnp
from jax import lax
from jax.experimental import pallas as pl
from jax.experimental.pallas import tpu as pltpu
from jax.experimental.pallas import tpu_sc as plsc

BATCH = 16384
N_FIELDS = 26
CARD = 64
OUT_W = N_FIELDS * CARD

NUM_WORKERS = 32
ROWS_PER_WORKER = BATCH // NUM_WORKERS
CHUNK_ROWS = 32
CHUNKS = ROWS_PER_WORKER // CHUNK_ROWS
LANES = 16


def _body(x_hbm, out_hbm, x_vmem, buf0, buf1, sem0, sem1, semx):
    c = lax.axis_index("c")
    s = lax.axis_index("s")
    wid = s * 2 + c

    ones = jnp.full((LANES,), 1, jnp.int32)
    zeros = jnp.zeros((LANES,), jnp.int32)

    def _splat(v):
        return jnp.full((LANES,), v, jnp.int32)

    xcopy = pltpu.make_async_copy(
        x_hbm.at[:, pl.ds(wid * ROWS_PER_WORKER, ROWS_PER_WORKER)], x_vmem, semx
    )
    xcopy.start()

    rows0 = lax.iota(jnp.int32, LANES)
    rows1 = rows0 + _splat(LANES)

    def scatter_pass(buf, k, value_vec):
        row0 = k * CHUNK_ROWS

        def ib(f, _):
            colbase = _splat(f * CARD)
            vals0 = x_vmem[f, pl.ds(row0, LANES)]
            plsc.store_scatter(buf, [rows0, colbase + vals0], value_vec)
            vals1 = x_vmem[f, pl.ds(row0 + LANES, LANES)]
            plsc.store_scatter(buf, [rows1, colbase + vals1], value_vec)
            return 0

        lax.fori_loop(0, N_FIELDS, ib, 0)

    rbase = wid * ROWS_PER_WORKER

    def dma(buf, k, sem):
        return pltpu.make_async_copy(
            buf, out_hbm.at[pl.ds(rbase + k * CHUNK_ROWS, CHUNK_ROWS)], sem
        )

    def zero_buf(buf):
        def z(r, _):
            for u in range(OUT_W // LANES):
                buf[r, pl.ds(u * LANES, LANES)] = zeros
            return 0

        lax.fori_loop(0, CHUNK_ROWS, z, 0)

    zero_buf(buf0)
    xcopy.wait()
    scatter_pass(buf0, 0, ones)
    dma(buf0, 0, sem0).start()
    zero_buf(buf1)
    scatter_pass(buf1, 1, ones)
    dma(buf1, 1, sem1).start()

    def chunk_pair(j, _):
        k0 = 2 * j
        k1 = 2 * j + 1
        dma(buf0, k0 - 2, sem0).wait()
        scatter_pass(buf0, k0 - 2, zeros)
        scatter_pass(buf0, k0, ones)
        dma(buf0, k0, sem0).start()
        dma(buf1, k1 - 2, sem1).wait()
        scatter_pass(buf1, k1 - 2, zeros)
        scatter_pass(buf1, k1, ones)
        dma(buf1, k1, sem1).start()
        return 0

    lax.fori_loop(1, CHUNKS // 2, chunk_pair, 0)
    dma(buf0, CHUNKS - 2, sem0).wait()
    dma(buf1, CHUNKS - 1, sem1).wait()


@jax.jit
def _onehot(x):
    xt = x.T
    mesh = plsc.VectorSubcoreMesh(core_axis_name="c", subcore_axis_name="s")
    f = functools.partial(
        pl.kernel,
        out_type=jax.ShapeDtypeStruct((BATCH, OUT_W), jnp.int32),
        scratch_types=[
            pltpu.VMEM((N_FIELDS, ROWS_PER_WORKER), jnp.int32),
            pltpu.VMEM((CHUNK_ROWS, OUT_W), jnp.int32),
            pltpu.VMEM((CHUNK_ROWS, OUT_W), jnp.int32),
            pltpu.SemaphoreType.DMA,
            pltpu.SemaphoreType.DMA,
            pltpu.SemaphoreType.DMA,
        ],
        mesh=mesh,
        compiler_params=pltpu.CompilerParams(needs_layout_passes=False),
    )(_body)
    return f(xt)


def kernel(x):
    return _onehot(x)
